# Initial kernel scaffold; baseline (speedup 1.0000x reference)
import jax
import jax.numpy as jnp
from jax.experimental import pallas as pl


def kernel(x, edge_attr, edge_index, batch, W1, b1, W2, b2, W3, b3, W4, b4, conv1_w, conv1_b, conv2_w, conv2_b, out_W, out_b, h1_W, h1_b, h2_W, h2_b):
    raise NotImplementedError("write your pallas kernel here")


# speedup vs baseline: 1.1418x; 1.1418x over previous
import jax
import jax.numpy as jnp
from jax.experimental import pallas as pl

N_GRAPHS = 64
K_SORT = 20
TOTAL_LATENT = 97


def _gcn(h, src, dst, W, b, dinv, norm_e):
    h = h @ W
    agg = jax.ops.segment_sum(norm_e[:, None] * h[src], dst, num_segments=h.shape[0])
    return agg + dinv[:, None] ** 2 * h + b


def _head_kernel(dense_ref, out_W_ref, out_b_ref, h1_W_ref, h1_b_ref,
                 h2_W_ref, h2_b_ref, o_ref):
    dense = dense_ref[...]
    fp = jnp.maximum(dense @ out_W_ref[...] + out_b_ref[...][None, :], 0.0)
    h1 = jnp.maximum(fp @ h1_W_ref[...] + h1_b_ref[...][None, :], 0.0)
    logits = h1 @ h2_W_ref[...] + h2_b_ref[...][None, :]
    m = jnp.max(logits, axis=1, keepdims=True)
    s = logits - m
    lse = jnp.log(jnp.sum(jnp.exp(s), axis=1, keepdims=True))
    o_ref[...] = s - lse


def kernel(x, edge_attr, edge_index, batch, W1, b1, W2, b2, W3, b3, W4, b4,
           conv1_w, conv1_b, conv2_w, conv2_b, out_W, out_b, h1_W, h1_b,
           h2_W, h2_b):
    N = x.shape[0]
    row, col = edge_index[0], edge_index[1]
    x_edge = jax.ops.segment_sum(edge_attr, row, num_segments=N)
    h = jnp.concatenate([x, x_edge], axis=1)

    deg = 1.0 + jax.ops.segment_sum(jnp.ones_like(col, jnp.float32), col,
                                    num_segments=N)
    dinv = jnp.where(deg > 0, deg ** -0.5, 0.0)
    norm_e = dinv[row] * dinv[col]

    o1 = _gcn(h, row, col, W1, b1, dinv, norm_e)
    o2 = _gcn(o1, row, col, W2, b2, dinv, norm_e)
    o3 = _gcn(o2, row, col, W3, b3, dinv, norm_e)
    o4 = _gcn(o3, row, col, W4, b4, dinv, norm_e)
    xcat = jnp.concatenate([o1, o2, o3, o4], axis=1)

    graph_ids = jnp.arange(N_GRAPHS, dtype=batch.dtype)
    scores = jnp.where(batch[None, :] == graph_ids[:, None], o4[:, 0][None, :],
                       -jnp.inf)
    vals, idx = jax.lax.top_k(scores, K_SORT)
    sp = xcat[idx]
    valid = vals != -jnp.inf
    bsp = jnp.where(valid[:, :, None], sp, jnp.zeros((), sp.dtype))
    to_conv = bsp.reshape(N_GRAPHS, 1, K_SORT * TOTAL_LATENT)
    c1 = jax.nn.relu(jax.lax.conv_general_dilated(
        to_conv, conv1_w, (TOTAL_LATENT,), 'VALID',
        dimension_numbers=('NCH', 'OIH', 'NCH')) + conv1_b[None, :, None])
    c1 = c1.reshape(N_GRAPHS, 16, c1.shape[-1] // 2, 2).max(axis=-1)
    c2 = jax.nn.relu(jax.lax.conv_general_dilated(
        c1, conv2_w, (1,), 'VALID',
        dimension_numbers=('NCH', 'OIH', 'NCH')) + conv2_b[None, :, None])
    dense = c2.reshape(N_GRAPHS, -1)

    return pl.pallas_call(
        _head_kernel,
        out_shape=jax.ShapeDtypeStruct((N_GRAPHS, h2_W.shape[1]), jnp.float32),
    )(dense, out_W, out_b, h1_W, h1_b, h2_W, h2_b)


# speedup vs baseline: 1.3338x; 1.1682x over previous
import jax
import jax.numpy as jnp
from jax import lax
from jax.experimental import pallas as pl
from jax.experimental.pallas import tpu as pltpu
from jax.experimental.pallas import tpu_sc as plsc

N_NODES = 10000
N_EDGES = 320000
N_GRAPHS = 64
K_SORT = 20
TOTAL_LATENT = 97

_NC = 2
_NS = 16
_NW = _NC * _NS
_C = 128
_L = N_EDGES + N_NODES
_LP = ((_L + _C - 1) // _C) * _C
_NCH = _LP // _C


def _pad_idx(ls):
    return jnp.concatenate([ls, jnp.zeros((_LP - _L,), ls.dtype)])


def _sc_gather32_body(table_hbm, idx_hbm, out_hbm, idx_v, rows_v, sem):
    c = lax.axis_index("c")
    s = lax.axis_index("s")
    w = s * _NC + c
    nchunks = jnp.where(w < (_NCH % _NW), _NCH // _NW + 1, _NCH // _NW)

    def chunk(k, carry):
        base = (k * _NW + w) * _C
        pltpu.sync_copy(idx_hbm.at[pl.ds(base, _C)], idx_v)
        pltpu.async_copy(table_hbm.at[idx_v], rows_v, sem).wait()
        pltpu.sync_copy(rows_v, out_hbm.at[pl.ds(base, _C), :])
        return carry

    lax.fori_loop(0, nchunks, chunk, 0)


@jax.jit
def _sc_gather32(table, idx_padded):
    fn = pl.kernel(
        _sc_gather32_body,
        out_type=jax.ShapeDtypeStruct((_LP, 32), jnp.float32),
        mesh=plsc.VectorSubcoreMesh(core_axis_name="c", subcore_axis_name="s"),
        scratch_types=[
            pltpu.VMEM((_C,), jnp.int32),
            pltpu.VMEM((_C, 32), jnp.float32),
            pltpu.SemaphoreType.DMA,
        ],
        compiler_params=pltpu.CompilerParams(use_tc_tiling_on_sc=False, needs_layout_passes=False),
    )
    return fn(table, idx_padded)


def _sc_gather1_body(table_hbm, idx_hbm, out_hbm, tab_v, idx_v, o_v, sem):
    c = lax.axis_index("c")
    s = lax.axis_index("s")
    w = s * _NC + c
    pltpu.sync_copy(table_hbm, tab_v)
    nchunks = jnp.where(w < (_NCH % _NW), _NCH // _NW + 1, _NCH // _NW)

    def chunk(k, carry):
        base = (k * _NW + w) * _C
        pltpu.sync_copy(idx_hbm.at[pl.ds(base, _C)], idx_v)
        for g in range(_C // 16):
            idxr = idx_v[pl.ds(g * 16, 16)]
            o_v[pl.ds(g * 16, 16)] = plsc.load_gather(tab_v, [idxr])
        pltpu.sync_copy(o_v, out_hbm.at[pl.ds(base, _C)])
        return carry

    lax.fori_loop(0, nchunks, chunk, 0)


@jax.jit
def _sc_gather1(table, idx_padded):
    fn = pl.kernel(
        _sc_gather1_body,
        out_type=jax.ShapeDtypeStruct((_LP,), jnp.float32),
        mesh=plsc.VectorSubcoreMesh(core_axis_name="c", subcore_axis_name="s"),
        scratch_types=[
            pltpu.VMEM((N_NODES,), jnp.float32),
            pltpu.VMEM((_C,), jnp.int32),
            pltpu.VMEM((_C,), jnp.float32),
            pltpu.SemaphoreType.DMA,
        ],
        compiler_params=pltpu.CompilerParams(use_tc_tiling_on_sc=False, needs_layout_passes=False),
    )
    return fn(table, idx_padded)


def _head_kernel(dense_ref, out_W_ref, out_b_ref, h1_W_ref, h1_b_ref,
                 h2_W_ref, h2_b_ref, o_ref):
    dense = dense_ref[...]
    fp = jnp.maximum(dense @ out_W_ref[...] + out_b_ref[...][None, :], 0.0)
    h1 = jnp.maximum(fp @ h1_W_ref[...] + h1_b_ref[...][None, :], 0.0)
    logits = h1 @ h2_W_ref[...] + h2_b_ref[...][None, :]
    m = jnp.max(logits, axis=1, keepdims=True)
    sh = logits - m
    lse = jnp.log(jnp.sum(jnp.exp(sh), axis=1, keepdims=True))
    o_ref[...] = sh - lse


def kernel(x, edge_attr, edge_index, batch, W1, b1, W2, b2, W3, b3, W4, b4,
           conv1_w, conv1_b, conv2_w, conv2_b, out_W, out_b, h1_W, h1_b,
           h2_W, h2_b):
    N = x.shape[0]
    row, col = edge_index[0], edge_index[1]

    x_edge = jax.ops.segment_sum(edge_attr, row, num_segments=N)
    h = jnp.concatenate([x, x_edge], axis=1)
    ar = jnp.arange(N, dtype=row.dtype)
    ls = jnp.concatenate([row, ar])
    ld = jnp.concatenate([col, ar])
    deg = jax.ops.segment_sum(jnp.ones(ld.shape[0], h.dtype), ld,
                              num_segments=N)
    dinv = jnp.where(deg > 0, deg ** -0.5, 0.0)
    norm = dinv[ls] * dinv[ld]
    ls_p = _pad_idx(ls)

    def gcn_wide(hcur, W, b):
        hw = hcur @ W
        g = _sc_gather32(hw, ls_p)[:_L]
        return jax.ops.segment_sum(norm[:, None] * g, ld, num_segments=N) + b

    o1 = gcn_wide(h, W1, b1)
    o2 = gcn_wide(o1, W2, b2)
    o3 = gcn_wide(o2, W3, b3)
    hw4 = (o3 @ W4)[:, 0]
    g4 = _sc_gather1(hw4, ls_p)[:_L]
    o4 = jax.ops.segment_sum((norm * g4)[:, None], ld, num_segments=N) + b4

    xcat = jnp.concatenate([o1, o2, o3, o4], axis=1)

    graph_ids = jnp.arange(N_GRAPHS, dtype=batch.dtype)
    scores = jnp.where(batch[None, :] == graph_ids[:, None], o4[:, 0][None, :],
                       -jnp.inf)
    vals, idx = jax.lax.top_k(scores, K_SORT)
    sp = xcat[idx]
    valid = vals != -jnp.inf
    bsp = jnp.where(valid[:, :, None], sp, jnp.zeros((), sp.dtype))
    to_conv = bsp.reshape(N_GRAPHS, 1, K_SORT * TOTAL_LATENT)
    c1 = jax.nn.relu(jax.lax.conv_general_dilated(
        to_conv, conv1_w, (TOTAL_LATENT,), 'VALID',
        dimension_numbers=('NCH', 'OIH', 'NCH')) + conv1_b[None, :, None])
    c1 = c1.reshape(N_GRAPHS, 16, c1.shape[-1] // 2, 2).max(axis=-1)
    c2 = jax.nn.relu(jax.lax.conv_general_dilated(
        c1, conv2_w, (1,), 'VALID',
        dimension_numbers=('NCH', 'OIH', 'NCH')) + conv2_b[None, :, None])
    dense = c2.reshape(N_GRAPHS, -1)

    return pl.pallas_call(
        _head_kernel,
        out_shape=jax.ShapeDtypeStruct((N_GRAPHS, h2_W.shape[1]), jnp.float32),
    )(dense, out_W, out_b, h1_W, h1_b, h2_W, h2_b)
